# baseline (speedup 1.0000x reference)
import functools

import jax
import jax.numpy as jnp
from jax.experimental import pallas as pl
from jax.experimental.pallas import tpu as pltpu

B, T, V, D, H = 2048, 32, 100, 128, 256
VP = 128


def _gru_kernel(idx_ref, h0_ref, len_ref, emb_ref, wihT_ref, whhT_ref,
                bih_ref, bhh_ref, out_ref):
    Graw = jnp.dot(emb_ref[...].astype(jnp.bfloat16),
                   wihT_ref[...].astype(jnp.bfloat16),
                   preferred_element_type=jnp.float32)
    whhT = whhT_ref[...]
    bih = bih_ref[...]
    bhh = bhh_ref[...]
    H2 = 2 * H
    row_emb = jnp.concatenate(
        [Graw[:, :H2], Graw[:, H2:], jnp.zeros((VP, H), jnp.float32)], axis=1)
    row_h = jnp.concatenate(
        [whhT[:, :H2], jnp.zeros((H, H), jnp.float32), whhT[:, H2:]], axis=1)
    row_b = jnp.concatenate(
        [bih[:, :H2] + bhh[:, :H2], bih[:, H2:], bhh[:, H2:]], axis=1)
    rhs = jnp.concatenate([row_emb, row_h, row_b], axis=0).astype(jnp.bfloat16)

    lens = len_ref[...]
    idx = idx_ref[...]
    h = h0_ref[...]
    hb = h.astype(jnp.bfloat16)
    BB = h.shape[0]
    iota_v = jax.lax.broadcasted_iota(jnp.int32, (1, VP), 1)
    ones_col = jnp.ones((BB, 1), jnp.bfloat16)

    for t in range(T):
        onehot = (idx[:, t][:, None] == iota_v).astype(jnp.bfloat16)
        lhs = jnp.concatenate([onehot, hb, ones_col], axis=1)
        gates = jnp.dot(lhs, rhs, preferred_element_type=jnp.float32)
        rz = jax.nn.sigmoid(gates[:, :H2])
        r = rz[:, :H]
        z = rz[:, H:]
        n = jnp.tanh(gates[:, H2:3 * H] + r * gates[:, 3 * H:])
        h = n + z * (h - n)
        hb = h.astype(jnp.bfloat16)
        out_ref[:, t, :] = jnp.where(t < lens, h, 0.0)


@functools.partial(jax.jit, static_argnames=("interpret",))
def _run(output, h0, lens2d, embp, wihT, whhT, bih2d, bhh2d, interpret=False):
    BB = 256
    grid = (B // BB,)
    return pl.pallas_call(
        _gru_kernel,
        grid=grid,
        in_specs=[
            pl.BlockSpec((BB, T), lambda i: (i, 0)),
            pl.BlockSpec((BB, H), lambda i: (i, 0)),
            pl.BlockSpec((BB, 1), lambda i: (i, 0)),
            pl.BlockSpec((VP, D), lambda i: (0, 0)),
            pl.BlockSpec((D, 3 * H), lambda i: (0, 0)),
            pl.BlockSpec((H, 3 * H), lambda i: (0, 0)),
            pl.BlockSpec((1, 3 * H), lambda i: (0, 0)),
            pl.BlockSpec((1, 3 * H), lambda i: (0, 0)),
        ],
        out_specs=pl.BlockSpec((BB, T, H), lambda i: (i, 0, 0)),
        out_shape=jax.ShapeDtypeStruct((B, T, H), jnp.float32),
        compiler_params=pltpu.CompilerParams(
            dimension_semantics=("parallel",)),
        interpret=interpret,
    )(output, h0, lens2d, embp, wihT, whhT, bih2d, bhh2d)


def kernel(output, conditioning, output_mask, output_word_len, emb,
           W_ih, W_hh, b_ih, b_hh, interpret=False):
    h0 = conditioning[0]
    lens2d = jnp.maximum(output_word_len, 1)[:, None].astype(jnp.int32)
    embp = jnp.concatenate([emb, jnp.zeros((VP - V, D), emb.dtype)], axis=0)
    return _run(output.astype(jnp.int32), h0, lens2d, embp,
                W_ih.T, W_hh.T, b_ih[None, :], b_hh[None, :],
                interpret=interpret)
